# baseline (speedup 1.0000x reference)
import jax
import jax.numpy as jnp
from jax.experimental import pallas as pl

N_NODES = 10000
N_EDGES = 320000
HID = 128
NB = 8
NELEM = 5
NLAYERS = 2
RMAX = 5.0
AVG_NEIGH = 32.0
ZS = jnp.array([1, 6, 7, 8, 16])
L_OF = jnp.array([0, 1, 1, 1, 2, 2, 2, 2, 2])


def _sph_harm(u):
    x, y, z = u[:, 0], u[:, 1], u[:, 2]
    s3 = jnp.sqrt(3.0)
    s15 = jnp.sqrt(15.0)
    s5 = jnp.sqrt(5.0)
    return jnp.stack([
        jnp.ones_like(x),
        s3 * x, s3 * y, s3 * z,
        s15 * x * y, s15 * y * z, (s5 / 2.0) * (3.0 * z * z - 1.0),
        s15 * x * z, (s15 / 2.0) * (x * x - y * y)
    ], axis=-1)


def _bessel_cutoff(r):
    n = jnp.arange(1, NB + 1, dtype=jnp.float32)
    rb = r[:, None]
    bess = jnp.sqrt(2.0 / RMAX) * jnp.sin(n * jnp.pi * rb / RMAX) / (rb + 1e-9)
    p = 6.0
    u = jnp.clip(r / RMAX, 0.0, 1.0)
    fc = (1.0 - (p + 1.0) * (p + 2.0) / 2.0 * u ** p
          + p * (p + 2.0) * u ** (p + 1.0)
          - p * (p + 1.0) / 2.0 * u ** (p + 2.0))
    return bess * fc[:, None]


def kernel(positions, atomic_numbers, edge_index, W_node_embed, W_up,
           W1, W2, W3, W_lin, W_prod):
    perm = jnp.argsort(edge_index[1])
    sender = edge_index[0][perm]
    receiver = edge_index[1][perm]
    vec = positions[receiver] - positions[sender]
    lengths = jnp.sqrt(jnp.sum(vec * vec, axis=-1) + 1e-12)
    u = vec / (lengths[:, None] + 1e-9)
    edge_attrs = _sph_harm(u)
    idx = jnp.searchsorted(ZS, atomic_numbers)
    node_attrs = jax.nn.one_hot(idx, NELEM, dtype=positions.dtype)
    edge_feats = _bessel_cutoff(lengths)
    node_feats = node_attrs @ W_node_embed
    N = positions.shape[0]
    for i in range(NLAYERS):
        h = node_feats @ W_up[i]
        t = jax.nn.silu(edge_feats @ W1[i])
        t = jax.nn.silu(t @ W2[i])
        tp_w = (t @ W3[i]).reshape(-1, 3, HID)
        mji = tp_w[:, L_OF, :] * h[sender][:, None, :] * edge_attrs[:, :, None]
        message = jax.ops.segment_sum(mji, receiver, num_segments=N,
                                      indices_are_sorted=True) / AVG_NEIGH
        lin = jnp.einsum('nlc,lcd->nld', message, W_lin[i][L_OF])
        scal = lin[:, 0, :]
        inv1 = jnp.sum(lin[:, 1:4, :] ** 2, axis=1)
        inv2 = jnp.sum(lin[:, 4:9, :] ** 2, axis=1)
        feat = scal + inv1 + inv2
        tmp = jnp.einsum('nc,ecd->ned', feat, W_prod[i])
        node_feats = jnp.einsum('ned,ne->nd', tmp, node_attrs)
    return node_feats


# speedup vs baseline: 6.9006x; 6.9006x over previous
import functools

import jax
import jax.numpy as jnp
from jax import lax
from jax.experimental import pallas as pl
from jax.experimental.pallas import tpu as pltpu
from jax.experimental.pallas import tpu_sc as plsc

N_NODES = 10000
N_EDGES = 320000
HID = 128
NB = 8
NELEM = 5
RMAX = 5.0
AVG_NEIGH = 32.0
L_OF = (0, 1, 1, 1, 2, 2, 2, 2, 2)

T = 512
BN = 128
NBLK = 80
NP = NBLK * BN
NCHUNK = N_EDGES // T + NBLK
EP = NCHUNK * T

NW = 32
PER_W = EP // NW
BC = 240
NCH_SC = PER_W // BC

TN = 512

_INTERPRET = False


def _sc_gather(table, idx, ncols):
    mesh = plsc.VectorSubcoreMesh(core_axis_name="c", subcore_axis_name="s")

    @functools.partial(
        pl.kernel, mesh=mesh,
        out_type=jax.ShapeDtypeStruct((EP, ncols), jnp.float32),
        scratch_types=[
            pltpu.VMEM((BC,), jnp.int32),
            pltpu.VMEM((BC, ncols), jnp.float32),
            pltpu.SemaphoreType.DMA,
        ],
    )
    def k(table_hbm, idx_hbm, out_hbm, idx_v, rows_v, sem):
        wid = lax.axis_index("s") * 2 + lax.axis_index("c")
        base = wid * PER_W

        def body(c, carry):
            off = base + c * BC
            pltpu.sync_copy(idx_hbm.at[pl.ds(off, BC)], idx_v)
            pltpu.async_copy(table_hbm.at[idx_v], rows_v, sem).wait()
            pltpu.sync_copy(rows_v, out_hbm.at[pl.ds(off, BC)])
            return carry

        lax.fori_loop(0, NCH_SC, body, 0)

    return k(table, idx)


def _node0_body(az_ref, wne_ref, wup_ref, na_ref, h1_ref):
    az = az_ref[...]
    cols = [(az == z).astype(jnp.float32) for z in (1, 6, 7, 8, 16)]
    na = jnp.concatenate(cols, axis=1)
    na_ref[...] = na
    nf0 = jnp.dot(na, wne_ref[...], preferred_element_type=jnp.float32)
    h1_ref[...] = jnp.dot(nf0, wup_ref[...], preferred_element_type=jnp.float32)


def _node0(az2d, w_ne, w_up0):
    return pl.pallas_call(
        _node0_body,
        grid=(NP // TN,),
        in_specs=[
            pl.BlockSpec((TN, 1), lambda i: (i, 0)),
            pl.BlockSpec((NELEM, HID), lambda i: (0, 0)),
            pl.BlockSpec((HID, HID), lambda i: (0, 0)),
        ],
        out_specs=[
            pl.BlockSpec((TN, NELEM), lambda i: (i, 0)),
            pl.BlockSpec((TN, HID), lambda i: (i, 0)),
        ],
        out_shape=[
            jax.ShapeDtypeStruct((NP, NELEM), jnp.float32),
            jax.ShapeDtypeStruct((NP, HID), jnp.float32),
        ],
        interpret=_INTERPRET,
    )(az2d, w_ne, w_up0)


def _msg_body(bid_ref, ps_ref, pr_ref, r_ref, hs_ref, w1_ref, w2_ref, w3_ref,
              out_ref):
    i = pl.program_id(0)
    b = bid_ref[i]

    vec = pr_ref[:, 0:3] - ps_ref[:, 0:3]
    r2 = jnp.sum(vec * vec, axis=1, keepdims=True) + 1e-12
    length = jnp.sqrt(r2)
    u = vec / (length + 1e-9)
    x, y, z = u[:, 0:1], u[:, 1:2], u[:, 2:3]
    s3 = jnp.sqrt(3.0)
    s15 = jnp.sqrt(15.0)
    s5 = jnp.sqrt(5.0)
    ys = [jnp.ones_like(x),
          s3 * x, s3 * y, s3 * z,
          s15 * x * y, s15 * y * z, (s5 / 2.0) * (3.0 * z * z - 1.0),
          s15 * x * z, (s15 / 2.0) * (x * x - y * y)]

    n = (lax.broadcasted_iota(jnp.int32, (1, NB), 1) + 1
         ).astype(jnp.float32)
    bess = jnp.sqrt(2.0 / RMAX) * jnp.sin(n * (jnp.pi / RMAX) * length) \
        / (length + 1e-9)
    uc = jnp.clip(length / RMAX, 0.0, 1.0)
    u6 = uc * uc * uc
    u6 = u6 * u6
    fc = 1.0 - 28.0 * u6 + 48.0 * u6 * uc - 21.0 * u6 * uc * uc
    ef = bess * fc

    t = ef @ w1_ref[...]
    t = t * jax.nn.sigmoid(t)
    t = t @ w2_ref[...]
    t = t * jax.nn.sigmoid(t)
    tp = jnp.dot(t, w3_ref[...], preferred_element_type=jnp.float32)

    hs = hs_ref[...]
    cols = []
    for l in range(9):
        p = L_OF[l]
        cols.append(tp[:, p * HID:(p + 1) * HID] * hs * ys[l])
    mji = jnp.concatenate(cols, axis=1)

    rloc = r_ref[...] - b * BN
    onehot_t = (rloc == lax.broadcasted_iota(jnp.int32, (1, BN), 1)
                ).astype(jnp.float32)
    contrib = lax.dot_general(
        onehot_t, mji, (((0,), (0,)), ((), ())),
        preferred_element_type=jnp.float32) * (1.0 / AVG_NEIGH)

    prev = bid_ref[jnp.maximum(i - 1, 0)]
    first = jnp.logical_or(i == 0, b != prev)

    @pl.when(first)
    def _():
        out_ref[...] = contrib

    @pl.when(jnp.logical_not(first))
    def _():
        out_ref[...] += contrib


def _msg(bid, ps, pr, r2d, hs, w1, w2, w3):
    grid_spec = pltpu.PrefetchScalarGridSpec(
        num_scalar_prefetch=1,
        grid=(NCHUNK,),
        in_specs=[
            pl.BlockSpec((T, HID), lambda i, bid: (i, 0)),
            pl.BlockSpec((T, HID), lambda i, bid: (i, 0)),
            pl.BlockSpec((T, 1), lambda i, bid: (i, 0)),
            pl.BlockSpec((T, HID), lambda i, bid: (i, 0)),
            pl.BlockSpec((NB, 64), lambda i, bid: (0, 0)),
            pl.BlockSpec((64, 64), lambda i, bid: (0, 0)),
            pl.BlockSpec((64, 3 * HID), lambda i, bid: (0, 0)),
        ],
        out_specs=pl.BlockSpec((BN, 9 * HID), lambda i, bid: (bid[i], 0)),
    )
    return pl.pallas_call(
        _msg_body,
        grid_spec=grid_spec,
        out_shape=jax.ShapeDtypeStruct((NP, 9 * HID), jnp.float32),
        interpret=_INTERPRET,
    )(bid, ps, pr, r2d, hs, w1, w2, w3)


def _node_body(msg_ref, na_ref, wlin_ref, wprod_ref, *rest):
    has_next = len(rest) == 3
    if has_next:
        wupn_ref, nf_ref, h_ref = rest
    else:
        nf_ref, = rest

    m = msg_ref[...]
    feat = None
    for l in range(9):
        p = L_OF[l]
        lin = jnp.dot(m[:, l * HID:(l + 1) * HID], wlin_ref[p],
                      preferred_element_type=jnp.float32)
        contrib = lin if l == 0 else lin * lin
        feat = contrib if feat is None else feat + contrib

    na = na_ref[...]
    out = None
    for e in range(NELEM):
        term = na[:, e:e + 1] * jnp.dot(feat, wprod_ref[e],
                                        preferred_element_type=jnp.float32)
        out = term if out is None else out + term
    nf_ref[...] = out
    if has_next:
        h_ref[...] = jnp.dot(out, wupn_ref[...],
                             preferred_element_type=jnp.float32)


def _node(msg, na, w_lin_i, w_prod_i, w_up_next):
    has_next = w_up_next is not None
    in_specs = [
        pl.BlockSpec((TN, 9 * HID), lambda i: (i, 0)),
        pl.BlockSpec((TN, NELEM), lambda i: (i, 0)),
        pl.BlockSpec((3, HID, HID), lambda i: (0, 0, 0)),
        pl.BlockSpec((NELEM, HID, HID), lambda i: (0, 0, 0)),
    ]
    args = [msg, na, w_lin_i, w_prod_i]
    out_specs = [pl.BlockSpec((TN, HID), lambda i: (i, 0))]
    out_shape = [jax.ShapeDtypeStruct((NP, HID), jnp.float32)]
    if has_next:
        in_specs.append(pl.BlockSpec((HID, HID), lambda i: (0, 0)))
        args.append(w_up_next)
        out_specs.append(pl.BlockSpec((TN, HID), lambda i: (i, 0)))
        out_shape.append(jax.ShapeDtypeStruct((NP, HID), jnp.float32))
    res = pl.pallas_call(
        _node_body,
        grid=(NP // TN,),
        in_specs=in_specs,
        out_specs=out_specs if has_next else out_specs[0],
        out_shape=out_shape if has_next else out_shape[0],
        interpret=_INTERPRET,
    )(*args)
    return res if has_next else (res, None)


def _build_indices(edge_index):
    sender = edge_index[0]
    receiver = edge_index[1]
    perm = jnp.argsort(receiver)
    r_sorted = receiver[perm]
    bstart = jnp.searchsorted(r_sorted,
                              jnp.arange(NBLK + 1, dtype=jnp.int32) * BN
                              ).astype(jnp.int32)
    cnt = bstart[1:] - bstart[:-1]
    nch = jnp.maximum(1, (cnt + T - 1) // T)
    cstart = jnp.concatenate([jnp.zeros((1,), jnp.int32),
                              jnp.cumsum(nch).astype(jnp.int32)])
    ci = jnp.arange(NCHUNK, dtype=jnp.int32)
    bid = jnp.clip(jnp.searchsorted(cstart, ci, side='right') - 1,
                   0, NBLK - 1).astype(jnp.int32)

    k = jnp.arange(EP, dtype=jnp.int32)
    ik = k // T
    bk = bid[ik]
    rank = (ik - cstart[bk]) * T + (k % T)
    j = bstart[bk] + rank
    valid = rank < cnt[bk]
    j_c = jnp.where(valid, j, 0)
    e_id = perm[j_c]
    spread = k % N_NODES
    s_pad = jnp.where(valid, sender[e_id], spread).astype(jnp.int32)
    rg_pad = jnp.where(valid, r_sorted[j_c], spread).astype(jnp.int32)
    r_pad = jnp.where(valid, r_sorted[j_c], -1).astype(jnp.int32)
    return bid, s_pad, rg_pad, r_pad.reshape(EP, 1)


def kernel(positions, atomic_numbers, edge_index, W_node_embed, W_up,
           W1, W2, W3, W_lin, W_prod):
    bid, s_pad, rg_pad, r2d = _build_indices(edge_index)

    pos128 = jnp.pad(positions, ((0, 0), (0, HID - 3)))
    ps = _sc_gather(pos128, s_pad, HID)
    pr = _sc_gather(pos128, rg_pad, HID)

    az2d = jnp.pad(atomic_numbers.astype(jnp.int32),
                   (0, NP - N_NODES)).reshape(NP, 1)
    na, h1 = _node0(az2d, W_node_embed, W_up[0])

    hs1 = _sc_gather(h1, s_pad, HID)
    msg1 = _msg(bid, ps, pr, r2d, hs1, W1[0], W2[0], W3[0])
    nf1, h2 = _node(msg1, na, W_lin[0], W_prod[0], W_up[1])

    hs2 = _sc_gather(h2, s_pad, HID)
    msg2 = _msg(bid, ps, pr, r2d, hs2, W1[1], W2[1], W3[1])
    nf2, _ = _node(msg2, na, W_lin[1], W_prod[1], None)

    return nf2[:N_NODES]


# speedup vs baseline: 18.9686x; 2.7488x over previous
import functools

import jax
import jax.numpy as jnp
from jax import lax
from jax.experimental import pallas as pl
from jax.experimental.pallas import tpu as pltpu
from jax.experimental.pallas import tpu_sc as plsc

N_NODES = 10000
N_EDGES = 320000
HID = 128
NB = 8
NELEM = 5
RMAX = 5.0
AVG_NEIGH = 32.0
L_OF = (0, 1, 1, 1, 2, 2, 2, 2, 2)

T = 512
BN = 128
NBLK = 80
NP = NBLK * BN
NCHUNK = N_EDGES // T + NBLK
EP = NCHUNK * T

NW = 32
PER_W = EP // NW
BC = 240
NCH_SC = PER_W // BC

TN = 512

_INTERPRET = False


def _sc_gather(table, idx, ncols):
    mesh = plsc.VectorSubcoreMesh(core_axis_name="c", subcore_axis_name="s")

    @functools.partial(
        pl.kernel, mesh=mesh,
        out_type=jax.ShapeDtypeStruct((EP, ncols), jnp.float32),
        scratch_types=[
            pltpu.VMEM((BC,), jnp.int32),
            pltpu.VMEM((BC, ncols), jnp.float32),
            pltpu.SemaphoreType.DMA,
        ],
    )
    def k(table_hbm, idx_hbm, out_hbm, idx_v, rows_v, sem):
        wid = lax.axis_index("s") * 2 + lax.axis_index("c")
        base = wid * PER_W

        def body(c, carry):
            off = base + c * BC
            pltpu.sync_copy(idx_hbm.at[pl.ds(off, BC)], idx_v)
            pltpu.async_copy(table_hbm.at[idx_v], rows_v, sem).wait()
            pltpu.sync_copy(rows_v, out_hbm.at[pl.ds(off, BC)])
            return carry

        lax.fori_loop(0, NCH_SC, body, 0)

    return k(table, idx)


def _node0_body(az_ref, wne_ref, wup_ref, na_ref, h1_ref):
    az = az_ref[...]
    cols = [(az == z).astype(jnp.float32) for z in (1, 6, 7, 8, 16)]
    na = jnp.concatenate(cols, axis=1)
    na_ref[...] = na
    nf0 = jnp.dot(na, wne_ref[...], preferred_element_type=jnp.float32)
    h1_ref[...] = jnp.dot(nf0, wup_ref[...], preferred_element_type=jnp.float32)


def _node0(az2d, w_ne, w_up0):
    return pl.pallas_call(
        _node0_body,
        grid=(NP // TN,),
        in_specs=[
            pl.BlockSpec((TN, 1), lambda i: (i, 0)),
            pl.BlockSpec((NELEM, HID), lambda i: (0, 0)),
            pl.BlockSpec((HID, HID), lambda i: (0, 0)),
        ],
        out_specs=[
            pl.BlockSpec((TN, NELEM), lambda i: (i, 0)),
            pl.BlockSpec((TN, HID), lambda i: (i, 0)),
        ],
        out_shape=[
            jax.ShapeDtypeStruct((NP, NELEM), jnp.float32),
            jax.ShapeDtypeStruct((NP, HID), jnp.float32),
        ],
        interpret=_INTERPRET,
    )(az2d, w_ne, w_up0)


def _msg_body(bid_ref, ps_ref, pr_ref, r_ref, hs_ref, w1_ref, w2_ref, w3_ref,
              out_ref):
    i = pl.program_id(0)
    b = bid_ref[i]

    vec = pr_ref[:, 0:3] - ps_ref[:, 0:3]
    r2 = jnp.sum(vec * vec, axis=1, keepdims=True) + 1e-12
    length = jnp.sqrt(r2)
    u = vec / (length + 1e-9)
    x, y, z = u[:, 0:1], u[:, 1:2], u[:, 2:3]
    s3 = jnp.sqrt(3.0)
    s15 = jnp.sqrt(15.0)
    s5 = jnp.sqrt(5.0)
    ys = [jnp.ones_like(x),
          s3 * x, s3 * y, s3 * z,
          s15 * x * y, s15 * y * z, (s5 / 2.0) * (3.0 * z * z - 1.0),
          s15 * x * z, (s15 / 2.0) * (x * x - y * y)]

    n = (lax.broadcasted_iota(jnp.int32, (1, NB), 1) + 1
         ).astype(jnp.float32)
    bess = jnp.sqrt(2.0 / RMAX) * jnp.sin(n * (jnp.pi / RMAX) * length) \
        / (length + 1e-9)
    uc = jnp.clip(length / RMAX, 0.0, 1.0)
    u6 = uc * uc * uc
    u6 = u6 * u6
    fc = 1.0 - 28.0 * u6 + 48.0 * u6 * uc - 21.0 * u6 * uc * uc
    ef = bess * fc

    t = ef @ w1_ref[...]
    t = t * jax.nn.sigmoid(t)
    t = t @ w2_ref[...]
    t = t * jax.nn.sigmoid(t)
    tp = jnp.dot(t, w3_ref[...], preferred_element_type=jnp.float32)

    hs = hs_ref[...]
    cols = []
    for l in range(9):
        p = L_OF[l]
        cols.append(tp[:, p * HID:(p + 1) * HID] * hs * ys[l])
    mji = jnp.concatenate(cols, axis=1)

    rloc = r_ref[...] - b * BN
    onehot_t = (rloc == lax.broadcasted_iota(jnp.int32, (1, BN), 1)
                ).astype(jnp.float32)
    contrib = lax.dot_general(
        onehot_t, mji, (((0,), (0,)), ((), ())),
        preferred_element_type=jnp.float32) * (1.0 / AVG_NEIGH)

    prev = bid_ref[jnp.maximum(i - 1, 0)]
    first = jnp.logical_or(i == 0, b != prev)

    @pl.when(first)
    def _():
        out_ref[...] = contrib

    @pl.when(jnp.logical_not(first))
    def _():
        out_ref[...] += contrib


def _msg(bid, ps, pr, r2d, hs, w1, w2, w3):
    grid_spec = pltpu.PrefetchScalarGridSpec(
        num_scalar_prefetch=1,
        grid=(NCHUNK,),
        in_specs=[
            pl.BlockSpec((T, HID), lambda i, bid: (i, 0)),
            pl.BlockSpec((T, HID), lambda i, bid: (i, 0)),
            pl.BlockSpec((T, 1), lambda i, bid: (i, 0)),
            pl.BlockSpec((T, HID), lambda i, bid: (i, 0)),
            pl.BlockSpec((NB, 64), lambda i, bid: (0, 0)),
            pl.BlockSpec((64, 64), lambda i, bid: (0, 0)),
            pl.BlockSpec((64, 3 * HID), lambda i, bid: (0, 0)),
        ],
        out_specs=pl.BlockSpec((BN, 9 * HID), lambda i, bid: (bid[i], 0)),
    )
    return pl.pallas_call(
        _msg_body,
        grid_spec=grid_spec,
        out_shape=jax.ShapeDtypeStruct((NP, 9 * HID), jnp.float32),
        interpret=_INTERPRET,
    )(bid, ps, pr, r2d, hs, w1, w2, w3)


def _node_body(msg_ref, na_ref, wlin_ref, wprod_ref, *rest):
    has_next = len(rest) == 3
    if has_next:
        wupn_ref, nf_ref, h_ref = rest
    else:
        nf_ref, = rest

    m = msg_ref[...]
    feat = None
    for l in range(9):
        p = L_OF[l]
        lin = jnp.dot(m[:, l * HID:(l + 1) * HID], wlin_ref[p],
                      preferred_element_type=jnp.float32)
        contrib = lin if l == 0 else lin * lin
        feat = contrib if feat is None else feat + contrib

    na = na_ref[...]
    out = None
    for e in range(NELEM):
        term = na[:, e:e + 1] * jnp.dot(feat, wprod_ref[e],
                                        preferred_element_type=jnp.float32)
        out = term if out is None else out + term
    nf_ref[...] = out
    if has_next:
        h_ref[...] = jnp.dot(out, wupn_ref[...],
                             preferred_element_type=jnp.float32)


def _node(msg, na, w_lin_i, w_prod_i, w_up_next):
    has_next = w_up_next is not None
    in_specs = [
        pl.BlockSpec((TN, 9 * HID), lambda i: (i, 0)),
        pl.BlockSpec((TN, NELEM), lambda i: (i, 0)),
        pl.BlockSpec((3, HID, HID), lambda i: (0, 0, 0)),
        pl.BlockSpec((NELEM, HID, HID), lambda i: (0, 0, 0)),
    ]
    args = [msg, na, w_lin_i, w_prod_i]
    out_specs = [pl.BlockSpec((TN, HID), lambda i: (i, 0))]
    out_shape = [jax.ShapeDtypeStruct((NP, HID), jnp.float32)]
    if has_next:
        in_specs.append(pl.BlockSpec((HID, HID), lambda i: (0, 0)))
        args.append(w_up_next)
        out_specs.append(pl.BlockSpec((TN, HID), lambda i: (i, 0)))
        out_shape.append(jax.ShapeDtypeStruct((NP, HID), jnp.float32))
    res = pl.pallas_call(
        _node_body,
        grid=(NP // TN,),
        in_specs=in_specs,
        out_specs=out_specs if has_next else out_specs[0],
        out_shape=out_shape if has_next else out_shape[0],
        interpret=_INTERPRET,
    )(*args)
    return res if has_next else (res, None)


def _build_indices(edge_index):
    sender = edge_index[0]
    receiver = edge_index[1]
    r_sorted, s_sorted = lax.sort([receiver, sender], num_keys=1)
    bstart = jnp.searchsorted(r_sorted,
                              jnp.arange(NBLK + 1, dtype=jnp.int32) * BN
                              ).astype(jnp.int32)
    cnt = bstart[1:] - bstart[:-1]
    nch = jnp.maximum(1, (cnt + T - 1) // T)
    cstart = jnp.concatenate([jnp.zeros((1,), jnp.int32),
                              jnp.cumsum(nch).astype(jnp.int32)])
    ci = jnp.arange(NCHUNK, dtype=jnp.int32)
    bid = jnp.clip(jnp.searchsorted(cstart, ci, side='right') - 1,
                   0, NBLK - 1).astype(jnp.int32)

    local = ci - cstart[bid]
    jbase = bstart[bid] + local * T
    rem = jnp.clip(cnt[bid] - local * T, 0, T)

    off = jnp.arange(T, dtype=jnp.int32)
    j_full = jbase[:, None] + off[None, :]
    valid = (off[None, :] < rem[:, None]).reshape(EP)
    j_c = jnp.where(valid, j_full.reshape(EP), 0)

    spread = jnp.arange(EP, dtype=jnp.int32) % N_NODES
    s_pad = jnp.where(valid, s_sorted[j_c], spread).astype(jnp.int32)
    r_raw = r_sorted[j_c]
    rg_pad = jnp.where(valid, r_raw, spread).astype(jnp.int32)
    r_pad = jnp.where(valid, r_raw, -1).astype(jnp.int32)
    return bid, s_pad, rg_pad, r_pad.reshape(EP, 1)


def kernel(positions, atomic_numbers, edge_index, W_node_embed, W_up,
           W1, W2, W3, W_lin, W_prod):
    bid, s_pad, rg_pad, r2d = _build_indices(edge_index)

    pos128 = jnp.pad(positions, ((0, 0), (0, HID - 3)))
    ps = _sc_gather(pos128, s_pad, HID)
    pr = _sc_gather(pos128, rg_pad, HID)

    az2d = jnp.pad(atomic_numbers.astype(jnp.int32),
                   (0, NP - N_NODES)).reshape(NP, 1)
    na, h1 = _node0(az2d, W_node_embed, W_up[0])

    hs1 = _sc_gather(h1, s_pad, HID)
    msg1 = _msg(bid, ps, pr, r2d, hs1, W1[0], W2[0], W3[0])
    nf1, h2 = _node(msg1, na, W_lin[0], W_prod[0], W_up[1])

    hs2 = _sc_gather(h2, s_pad, HID)
    msg2 = _msg(bid, ps, pr, r2d, hs2, W1[1], W2[1], W3[1])
    nf2, _ = _node(msg2, na, W_lin[1], W_prod[1], None)

    return nf2[:N_NODES]
